# baseline (speedup 1.0000x reference)
import functools

import jax
import jax.numpy as jnp
from jax import lax
from jax.experimental import pallas as pl
from jax.experimental.pallas import tpu as pltpu
from jax.experimental.pallas import tpu_sc as plsc

H = W = 4096
C = 3
CHAN = H * W
NC, NS, L = 2, 16, 16
NW = NC * NS
NBINS = 256
HB = C * NBINS

BR = 8
CH = BR * W
HALF = CH // 2
BPC = (CHAN // NW) // CH
G = C * BPC
ROWS_PW = H // NW

_mesh = plsc.VectorSubcoreMesh(core_axis_name="c", subcore_axis_name="s")
_params = pltpu.CompilerParams(needs_layout_passes=False)


def _wid():
    return lax.axis_index("s") * NC + lax.axis_index("c")


def _band(wid, g):
    c = g >> 4
    row = wid * ROWS_PW + (g & (BPC - 1)) * BR
    return c, row


@functools.partial(
    pl.kernel,
    mesh=_mesh,
    compiler_params=_params,
    out_type=jax.ShapeDtypeStruct((NC * HB,), jnp.int32),
    scratch_types=[
        pltpu.VMEM((BR, W), jnp.float32),
        pltpu.VMEM((BR, W), jnp.float32),
        pltpu.VMEM((HALF,), jnp.int32),
        pltpu.VMEM((HALF,), jnp.int32),
        pltpu.VMEM((HALF,), jnp.int32),
        pltpu.VMEM((HB,), jnp.int32),
        pltpu.VMEM_SHARED((HB,), jnp.int32),
        pltpu.SemaphoreType.DMA,
        pltpu.SemaphoreType.DMA,
        pltpu.SemaphoreType.DMA,
        pltpu.SemaphoreType.DMA,
    ],
)
def _hist_kernel(x_hbm, out_hbm, in0, in1, bb0, bb1, onesb, lhist, shist,
                 sem0, sem1, ss0, ss1):
    wid = _wid()
    bufs = (in0, in1)
    sems = (sem0, sem1)
    bbufs = (bb0, bb1)
    ssems = (ss0, ss1)
    ones = jnp.ones((L,), jnp.int32)
    zeros = jnp.zeros((L,), jnp.int32)

    sid = lax.axis_index("s")

    def zero_body(i, _):
        lhist[pl.ds(i * L, L)] = zeros
        return 0

    lax.fori_loop(0, HB // L, zero_body, 0)

    @pl.when(sid == 0)
    def _():
        pltpu.sync_copy(lhist, shist)
    plsc.subcore_barrier()

    def ones_body(r, _):
        onesb[pl.ds(r * L, L)] = ones
        return 0

    lax.fori_loop(0, HALF // L, ones_body, 0)

    def in_copy(g, b):
        c, row = _band(wid, g)
        return pltpu.make_async_copy(
            x_hbm.at[0, c, pl.ds(row, BR), :], bufs[b], sems[b])

    def scat_copy(h):
        return pltpu.make_async_copy(onesb, shist.at[bbufs[h]], ssems[h])

    in_copy(0, 0).start()
    in_copy(1, 1).start()

    def chunk_body(p, _):
        for b in range(2):
            g = 2 * p + b
            coff = (g >> 4) << 8
            in_copy(g, b).wait()
            buf = bufs[b]

            for h in range(2):
                @pl.when(g >= 1)
                def _():
                    scat_copy(h).wait()

                bbuf = bbufs[h]
                for rr in range(BR // 2):
                    r = h * (BR // 2) + rr

                    @plsc.parallel_loop(0, W, step=L, unroll=8)
                    def _(i, buf=buf, r=r, rr=rr, bbuf=bbuf, coff=coff):
                        xv = buf[r, pl.ds(i, L)]
                        bn = (xv * 255.0).astype(jnp.int32) + coff
                        bbuf[pl.ds(rr * W + i, L)] = bn

                scat_copy(h).start(add=True)

            @pl.when(p < (G // 2) - 1)
            def _():
                in_copy(g + 2, b).start()
        return 0

    lax.fori_loop(0, G // 2, chunk_body, 0)
    scat_copy(0).wait()
    scat_copy(1).wait()
    plsc.subcore_barrier()

    @pl.when(sid == 0)
    def _():
        cid = lax.axis_index("c")
        pltpu.sync_copy(shist, out_hbm.at[pl.ds(cid * HB, HB)])


@functools.partial(
    pl.kernel,
    mesh=_mesh,
    compiler_params=_params,
    out_type=jax.ShapeDtypeStruct((1, C, H, W), jnp.float32),
    scratch_types=[
        pltpu.VMEM((NC * HB,), jnp.int32),
        pltpu.VMEM((HB,), jnp.int32),
        pltpu.VMEM((HB,), jnp.float32),
        pltpu.VMEM((NBINS,), jnp.float32),
        pltpu.VMEM((BR, W), jnp.float32),
        pltpu.VMEM((BR, W), jnp.float32),
        pltpu.VMEM((BR, W), jnp.float32),
        pltpu.SemaphoreType.DMA,
        pltpu.SemaphoreType.DMA,
        pltpu.SemaphoreType.DMA,
        pltpu.SemaphoreType.DMA,
        pltpu.SemaphoreType.DMA,
        pltpu.SemaphoreType.DMA,
    ],
)
def _apply_kernel(x_hbm, ph_hbm, out_hbm, pbuf, hsum, lut, lutc,
                  b0, b1, b2, si0, si1, si2, so0, so1, so2):
    wid = _wid()
    bufs = (b0, b1, b2)
    isems = (si0, si1, si2)
    osems = (so0, so1, so2)

    def in_copy(g, b):
        c, row = _band(wid, g)
        return pltpu.make_async_copy(
            x_hbm.at[0, c, pl.ds(row, BR), :], bufs[b], isems[b])

    def out_copy(g, b):
        c, row = _band(wid, g)
        return pltpu.make_async_copy(
            bufs[b], out_hbm.at[0, c, pl.ds(row, BR), :], osems[b])

    in_copy(0, 0).start()
    in_copy(1, 1).start()

    pltpu.sync_copy(ph_hbm, pbuf)

    def sum_body(j, _):
        acc = jnp.zeros((L,), jnp.int32)
        for w in range(NC):
            acc = acc + pbuf[pl.ds(w * HB + j * L, L)]
        hsum[pl.ds(j * L, L)] = acc
        return 0

    lax.fori_loop(0, HB // L, sum_body, 0)

    for c in range(C):
        carry = jnp.int32(0)
        cmin = jnp.int32(CHAN)
        for j in range(NBINS // L):
            v = hsum[pl.ds(c * NBINS + j * L, L)]
            cdf = jnp.cumsum(v) + carry
            hsum[pl.ds(c * NBINS + j * L, L)] = cdf
            carry = carry + jnp.sum(v)
            cmin = jnp.minimum(
                cmin, jnp.min(jnp.where(cdf > 0, cdf, jnp.int32(CHAN))))
        denom = jnp.maximum(jnp.int32(CHAN) - cmin, 1)
        denf = denom.astype(jnp.float32)
        for j in range(NBINS // L):
            cdf = hsum[pl.ds(c * NBINS + j * L, L)]
            kf = (cdf - cmin).astype(jnp.float32)
            f = jnp.maximum(kf / denf * 255.0, -1.0)
            t = f.astype(jnp.int32)
            fr = f - t.astype(jnp.float32)
            inc = jnp.where(fr > 0.5, jnp.int32(1),
                            jnp.where(fr == 0.5, t & 1, jnp.int32(0)))
            r = jnp.clip(t + inc, 0, 255)
            lut[pl.ds(c * NBINS + j * L, L)] = r.astype(jnp.float32) / 255.0

    def load_lutc(coff):
        def cp(j, _):
            lutc[pl.ds(j * L, L)] = lut[pl.ds(coff + j * L, L)]
            return 0
        lax.fori_loop(0, NBINS // L, cp, 0)

    def chunk_body(p, _):
        for b in range(3):
            g = 3 * p + b
            if b == 0:
                @pl.when(p >= 1)
                def _():
                    out_copy(g - 2, 1).wait()
                    in_copy(g + 1, 1).start()
            elif b == 1:
                @pl.when(p >= 1)
                def _():
                    out_copy(g - 2, 2).wait()
                in_copy(g + 1, 2).start()
            else:
                out_copy(g - 2, 0).wait()

                @pl.when(p < (G // 3) - 1)
                def _():
                    in_copy(g + 1, 0).start()

            @pl.when((g & (BPC - 1)) == 0)
            def _():
                load_lutc((g >> 4) << 8)

            in_copy(g, b).wait()
            buf = bufs[b]

            for r in range(BR):
                @plsc.parallel_loop(0, W, step=L, unroll=8)
                def _(i, buf=buf, r=r):
                    xv = buf[r, pl.ds(i, L)]
                    bn = (xv * 255.0).astype(jnp.int32)
                    buf[r, pl.ds(i, L)] = plsc.load_gather(lutc, [bn])

            out_copy(g, b).start()
        return 0

    lax.fori_loop(0, G // 3, chunk_body, 0)
    out_copy(G - 2, 1).wait()
    out_copy(G - 1, 2).wait()


def kernel(x):
    ph = _hist_kernel(x)
    return _apply_kernel(x, ph)


# speedup vs baseline: 2.2930x; 2.2930x over previous
import functools

import jax
import jax.numpy as jnp
from jax import lax
from jax.experimental import pallas as pl
from jax.experimental.pallas import tpu as pltpu
from jax.experimental.pallas import tpu_sc as plsc

H = W = 4096
C = 3
CHAN = H * W
NC, NS, L = 2, 16, 16
NW = NC * NS
NBINS = 256
HB = C * NBINS

BR = 8
CH = BR * W
CW = CH // 4
BPC = (CHAN // NW) // CH
G = C * BPC
ROWS_PW = H // NW
TOTAL4 = C * CHAN // 4

_mesh = plsc.VectorSubcoreMesh(core_axis_name="c", subcore_axis_name="s")
_params = pltpu.CompilerParams(needs_layout_passes=False)


def _wid():
    return lax.axis_index("s") * NC + lax.axis_index("c")


def _band(wid, g):
    c = g >> 4
    row = wid * ROWS_PW + (g & (BPC - 1)) * BR
    return c, row


@functools.partial(
    pl.kernel,
    mesh=_mesh,
    compiler_params=_params,
    out_type=(
        jax.ShapeDtypeStruct((NW * HB,), jnp.int32),
        jax.ShapeDtypeStruct((TOTAL4,), jnp.int32),
    ),
    scratch_types=[
        pltpu.VMEM((BR, W), jnp.float32),
        pltpu.VMEM((BR, W), jnp.float32),
        pltpu.VMEM((CW,), jnp.int32),
        pltpu.VMEM((CW,), jnp.int32),
        pltpu.VMEM((HB,), jnp.int32),
        pltpu.SemaphoreType.DMA,
        pltpu.SemaphoreType.DMA,
        pltpu.SemaphoreType.DMA,
        pltpu.SemaphoreType.DMA,
    ],
)
def _hist_kernel(x_hbm, ph_hbm, bins_hbm, in0, in1, bp0, bp1, lhist,
                 si0, si1, sb0, sb1):
    wid = _wid()
    bufs = (in0, in1)
    isems = (si0, si1)
    bpbufs = (bp0, bp1)
    bsems = (sb0, sb1)
    ones = jnp.ones((L,), jnp.int32)
    zeros = jnp.zeros((L,), jnp.int32)

    def zero_body(i, _):
        lhist[pl.ds(i * L, L)] = zeros
        return 0

    lax.fori_loop(0, HB // L, zero_body, 0)

    def in_copy(g, b):
        c, row = _band(wid, g)
        return pltpu.make_async_copy(
            x_hbm.at[0, c, pl.ds(row, BR), :], bufs[b], isems[b])

    def bins_copy(g, b):
        c, row = _band(wid, g)
        off = (c * H + row) * (W // 4)
        return pltpu.make_async_copy(
            bpbufs[b], bins_hbm.at[pl.ds(off, CW)], bsems[b])

    in_copy(0, 0).start()
    in_copy(1, 1).start()

    def chunk_body(p, _):
        for b in range(2):
            g = 2 * p + b
            coff = pl.multiple_of((g >> 4) << 8, NBINS)
            hist_c = lhist.at[pl.ds(coff, NBINS)]
            in_copy(g, b).wait()

            @pl.when(p >= 1)
            def _():
                bins_copy(g - 2, b).wait()

            buf = bufs[b]
            bpbuf = bpbufs[b]

            for r in range(BR):
                @plsc.parallel_loop(0, W, step=4 * L, unroll=2)
                def _(i, buf=buf, r=r, bpbuf=bpbuf, hist_c=hist_c):
                    b0 = (buf[r, pl.ds(i, L)] * 255.0).astype(jnp.int32)
                    b1 = (buf[r, pl.ds(i + L, L)] * 255.0).astype(jnp.int32)
                    b2 = (buf[r, pl.ds(i + 2 * L, L)] * 255.0).astype(
                        jnp.int32)
                    b3 = (buf[r, pl.ds(i + 3 * L, L)] * 255.0).astype(
                        jnp.int32)
                    plsc.addupdate_scatter(hist_c, [b0], ones)
                    plsc.addupdate_scatter(hist_c, [b1], ones)
                    plsc.addupdate_scatter(hist_c, [b2], ones)
                    plsc.addupdate_scatter(hist_c, [b3], ones)
                    w = (b0 | jnp.left_shift(b1, 8)
                         | jnp.left_shift(b2, 16) | jnp.left_shift(b3, 24))
                    bpbuf[pl.ds(r * (W // 4) + (i >> 2), L)] = w

            bins_copy(g, b).start()

            @pl.when(p < (G // 2) - 1)
            def _():
                in_copy(g + 2, b).start()
        return 0

    lax.fori_loop(0, G // 2, chunk_body, 0)
    bins_copy(G - 2, 0).wait()
    bins_copy(G - 1, 1).wait()
    pltpu.sync_copy(lhist, ph_hbm.at[pl.ds(wid * HB, HB)])


@functools.partial(
    pl.kernel,
    mesh=_mesh,
    compiler_params=_params,
    out_type=jax.ShapeDtypeStruct((1, C, H, W), jnp.float32),
    scratch_types=[
        pltpu.VMEM((NW * HB,), jnp.int32),
        pltpu.VMEM((HB,), jnp.int32),
        pltpu.VMEM((HB,), jnp.float32),
        pltpu.VMEM((NBINS,), jnp.float32),
        pltpu.VMEM((CW,), jnp.int32),
        pltpu.VMEM((CW,), jnp.int32),
        pltpu.VMEM((BR, W), jnp.float32),
        pltpu.VMEM((BR, W), jnp.float32),
        pltpu.SemaphoreType.DMA,
        pltpu.SemaphoreType.DMA,
        pltpu.SemaphoreType.DMA,
        pltpu.SemaphoreType.DMA,
    ],
)
def _apply_kernel(bins_hbm, ph_hbm, out_hbm, pbuf, hsum, lut, lutc,
                  ib0, ib1, ob0, ob1, si0, si1, so0, so1):
    wid = _wid()
    ibufs = (ib0, ib1)
    obufs = (ob0, ob1)
    isems = (si0, si1)
    osems = (so0, so1)

    def in_copy(g, b):
        c, row = _band(wid, g)
        off = (c * H + row) * (W // 4)
        return pltpu.make_async_copy(
            bins_hbm.at[pl.ds(off, CW)], ibufs[b], isems[b])

    def out_copy(g, b):
        c, row = _band(wid, g)
        return pltpu.make_async_copy(
            obufs[b], out_hbm.at[0, c, pl.ds(row, BR), :], osems[b])

    in_copy(0, 0).start()
    in_copy(1, 1).start()

    pltpu.sync_copy(ph_hbm, pbuf)

    def sum_body(j, _):
        acc = jnp.zeros((L,), jnp.int32)
        for w in range(NW):
            acc = acc + pbuf[pl.ds(w * HB + j * L, L)]
        hsum[pl.ds(j * L, L)] = acc
        return 0

    lax.fori_loop(0, HB // L, sum_body, 0)

    for c in range(C):
        carry = jnp.int32(0)
        cmin = jnp.int32(CHAN)
        for j in range(NBINS // L):
            v = hsum[pl.ds(c * NBINS + j * L, L)]
            cdf = jnp.cumsum(v) + carry
            hsum[pl.ds(c * NBINS + j * L, L)] = cdf
            carry = carry + jnp.sum(v)
            cmin = jnp.minimum(
                cmin, jnp.min(jnp.where(cdf > 0, cdf, jnp.int32(CHAN))))
        denom = jnp.maximum(jnp.int32(CHAN) - cmin, 1)
        denf = denom.astype(jnp.float32)
        for j in range(NBINS // L):
            cdf = hsum[pl.ds(c * NBINS + j * L, L)]
            kf = (cdf - cmin).astype(jnp.float32)
            f = jnp.maximum(kf / denf * 255.0, -1.0)
            t = f.astype(jnp.int32)
            fr = f - t.astype(jnp.float32)
            inc = jnp.where(fr > 0.5, jnp.int32(1),
                            jnp.where(fr == 0.5, t & 1, jnp.int32(0)))
            r = jnp.clip(t + inc, 0, 255)
            lut[pl.ds(c * NBINS + j * L, L)] = r.astype(jnp.float32) / 255.0

    def load_lutc(coff):
        def cp(j, _):
            lutc[pl.ds(j * L, L)] = lut[pl.ds(coff + j * L, L)]
            return 0
        lax.fori_loop(0, NBINS // L, cp, 0)

    def chunk_body(p, _):
        for b in range(2):
            g = 2 * p + b

            @pl.when((g & (BPC - 1)) == 0)
            def _():
                load_lutc((g >> 4) << 8)

            in_copy(g, b).wait()

            @pl.when(p >= 1)
            def _():
                out_copy(g - 2, b).wait()

            ibuf = ibufs[b]
            obuf = obufs[b]

            for r in range(BR):
                @plsc.parallel_loop(0, W // 4, step=L, unroll=4)
                def _(k, ibuf=ibuf, obuf=obuf, r=r):
                    wv = ibuf[pl.ds(r * (W // 4) + k, L)]
                    b0 = wv & 255
                    b1 = lax.shift_right_logical(wv, 8) & 255
                    b2 = lax.shift_right_logical(wv, 16) & 255
                    b3 = lax.shift_right_logical(wv, 24)
                    i0 = k * 4
                    obuf[r, pl.ds(i0, L)] = plsc.load_gather(lutc, [b0])
                    obuf[r, pl.ds(i0 + L, L)] = plsc.load_gather(lutc, [b1])
                    obuf[r, pl.ds(i0 + 2 * L, L)] = plsc.load_gather(
                        lutc, [b2])
                    obuf[r, pl.ds(i0 + 3 * L, L)] = plsc.load_gather(
                        lutc, [b3])

            out_copy(g, b).start()

            @pl.when(p < (G // 2) - 1)
            def _():
                in_copy(g + 2, b).start()
        return 0

    lax.fori_loop(0, G // 2, chunk_body, 0)
    out_copy(G - 2, 0).wait()
    out_copy(G - 1, 1).wait()


def kernel(x):
    ph, bins = _hist_kernel(x)
    return _apply_kernel(bins, ph)


# speedup vs baseline: 2.3837x; 1.0395x over previous
import functools

import jax
import jax.numpy as jnp
from jax import lax
from jax.experimental import pallas as pl
from jax.experimental.pallas import tpu as pltpu
from jax.experimental.pallas import tpu_sc as plsc

H = W = 4096
C = 3
CHAN = H * W
NC, NS, L = 2, 16, 16
NW = NC * NS
NBINS = 256
HB = C * NBINS

BR = 8
CH = BR * W
CW = CH // 4
BPC = (CHAN // NW) // CH
G = C * BPC
ROWS_PW = H // NW
TOTAL4 = C * CHAN // 4

_mesh = plsc.VectorSubcoreMesh(core_axis_name="c", subcore_axis_name="s")
_params = pltpu.CompilerParams(needs_layout_passes=False)


def _wid():
    return lax.axis_index("s") * NC + lax.axis_index("c")


def _band(wid, g):
    c = g >> 4
    row = wid * ROWS_PW + (g & (BPC - 1)) * BR
    return c, row


@functools.partial(
    pl.kernel,
    mesh=_mesh,
    compiler_params=_params,
    out_type=(
        jax.ShapeDtypeStruct((NW * HB,), jnp.int32),
        jax.ShapeDtypeStruct((TOTAL4,), jnp.int32),
    ),
    scratch_types=[
        pltpu.VMEM((BR, W), jnp.float32),
        pltpu.VMEM((BR, W), jnp.float32),
        pltpu.VMEM((CW,), jnp.int32),
        pltpu.VMEM((CW,), jnp.int32),
        pltpu.VMEM((HB,), jnp.int32),
        pltpu.SemaphoreType.DMA,
        pltpu.SemaphoreType.DMA,
        pltpu.SemaphoreType.DMA,
        pltpu.SemaphoreType.DMA,
    ],
)
def _hist_kernel(x_hbm, ph_hbm, bins_hbm, in0, in1, bp0, bp1, lhist,
                 si0, si1, sb0, sb1):
    wid = _wid()
    bufs = (in0, in1)
    isems = (si0, si1)
    bpbufs = (bp0, bp1)
    bsems = (sb0, sb1)
    ones = jnp.ones((L,), jnp.int32)
    zeros = jnp.zeros((L,), jnp.int32)

    def zero_body(i, _):
        lhist[pl.ds(i * L, L)] = zeros
        return 0

    lax.fori_loop(0, HB // L, zero_body, 0)

    def in_copy(g, b):
        c, row = _band(wid, g)
        return pltpu.make_async_copy(
            x_hbm.at[0, c, pl.ds(row, BR), :], bufs[b], isems[b])

    def bins_copy(g, b):
        c, row = _band(wid, g)
        off = (c * H + row) * (W // 4)
        return pltpu.make_async_copy(
            bpbufs[b], bins_hbm.at[pl.ds(off, CW)], bsems[b])

    in_copy(0, 0).start()
    in_copy(1, 1).start()

    def chunk_body(p, _):
        for b in range(2):
            g = 2 * p + b
            coff = pl.multiple_of((g >> 4) << 8, NBINS)
            hist_c = lhist.at[pl.ds(coff, NBINS)]
            in_copy(g, b).wait()

            @pl.when(p >= 1)
            def _():
                bins_copy(g - 2, b).wait()

            buf = bufs[b]
            bpbuf = bpbufs[b]

            for r in range(BR):
                @plsc.parallel_loop(0, W, step=4 * L, unroll=4)
                def _(i, buf=buf, r=r, bpbuf=bpbuf, hist_c=hist_c):
                    b0 = (buf[r, pl.ds(i, L)] * 255.0).astype(jnp.int32)
                    b1 = (buf[r, pl.ds(i + L, L)] * 255.0).astype(jnp.int32)
                    b2 = (buf[r, pl.ds(i + 2 * L, L)] * 255.0).astype(
                        jnp.int32)
                    b3 = (buf[r, pl.ds(i + 3 * L, L)] * 255.0).astype(
                        jnp.int32)
                    plsc.addupdate_scatter(hist_c, [b0], ones)
                    plsc.addupdate_scatter(hist_c, [b1], ones)
                    plsc.addupdate_scatter(hist_c, [b2], ones)
                    plsc.addupdate_scatter(hist_c, [b3], ones)
                    w = (b0 | jnp.left_shift(b1, 8)
                         | jnp.left_shift(b2, 16) | jnp.left_shift(b3, 24))
                    bpbuf[pl.ds(r * (W // 4) + (i >> 2), L)] = w

            bins_copy(g, b).start()

            @pl.when(p < (G // 2) - 1)
            def _():
                in_copy(g + 2, b).start()
        return 0

    lax.fori_loop(0, G // 2, chunk_body, 0)
    bins_copy(G - 2, 0).wait()
    bins_copy(G - 1, 1).wait()
    pltpu.sync_copy(lhist, ph_hbm.at[pl.ds(wid * HB, HB)])


@functools.partial(
    pl.kernel,
    mesh=_mesh,
    compiler_params=_params,
    out_type=jax.ShapeDtypeStruct((1, C, H, W), jnp.float32),
    scratch_types=[
        pltpu.VMEM((NW * HB,), jnp.int32),
        pltpu.VMEM((HB,), jnp.int32),
        pltpu.VMEM((HB,), jnp.float32),
        pltpu.VMEM((NBINS,), jnp.float32),
        pltpu.VMEM((CW,), jnp.int32),
        pltpu.VMEM((CW,), jnp.int32),
        pltpu.VMEM((BR, W), jnp.float32),
        pltpu.VMEM((BR, W), jnp.float32),
        pltpu.SemaphoreType.DMA,
        pltpu.SemaphoreType.DMA,
        pltpu.SemaphoreType.DMA,
        pltpu.SemaphoreType.DMA,
    ],
)
def _apply_kernel(bins_hbm, ph_hbm, out_hbm, pbuf, hsum, lut, lutc,
                  ib0, ib1, ob0, ob1, si0, si1, so0, so1):
    wid = _wid()
    ibufs = (ib0, ib1)
    obufs = (ob0, ob1)
    isems = (si0, si1)
    osems = (so0, so1)

    def in_copy(g, b):
        c, row = _band(wid, g)
        off = (c * H + row) * (W // 4)
        return pltpu.make_async_copy(
            bins_hbm.at[pl.ds(off, CW)], ibufs[b], isems[b])

    def out_copy(g, b):
        c, row = _band(wid, g)
        return pltpu.make_async_copy(
            obufs[b], out_hbm.at[0, c, pl.ds(row, BR), :], osems[b])

    in_copy(0, 0).start()
    in_copy(1, 1).start()

    pltpu.sync_copy(ph_hbm, pbuf)

    def sum_body(j, _):
        acc = jnp.zeros((L,), jnp.int32)
        for w in range(NW):
            acc = acc + pbuf[pl.ds(w * HB + j * L, L)]
        hsum[pl.ds(j * L, L)] = acc
        return 0

    lax.fori_loop(0, HB // L, sum_body, 0)

    for c in range(C):
        carry = jnp.int32(0)
        cmin = jnp.int32(CHAN)
        for j in range(NBINS // L):
            v = hsum[pl.ds(c * NBINS + j * L, L)]
            cdf = jnp.cumsum(v) + carry
            hsum[pl.ds(c * NBINS + j * L, L)] = cdf
            carry = carry + jnp.sum(v)
            cmin = jnp.minimum(
                cmin, jnp.min(jnp.where(cdf > 0, cdf, jnp.int32(CHAN))))
        denom = jnp.maximum(jnp.int32(CHAN) - cmin, 1)
        denf = denom.astype(jnp.float32)
        for j in range(NBINS // L):
            cdf = hsum[pl.ds(c * NBINS + j * L, L)]
            kf = (cdf - cmin).astype(jnp.float32)
            f = jnp.maximum(kf / denf * 255.0, -1.0)
            t = f.astype(jnp.int32)
            fr = f - t.astype(jnp.float32)
            inc = jnp.where(fr > 0.5, jnp.int32(1),
                            jnp.where(fr == 0.5, t & 1, jnp.int32(0)))
            r = jnp.clip(t + inc, 0, 255)
            lut[pl.ds(c * NBINS + j * L, L)] = r.astype(jnp.float32) / 255.0

    def load_lutc(coff):
        def cp(j, _):
            lutc[pl.ds(j * L, L)] = lut[pl.ds(coff + j * L, L)]
            return 0
        lax.fori_loop(0, NBINS // L, cp, 0)

    def chunk_body(p, _):
        for b in range(2):
            g = 2 * p + b

            @pl.when((g & (BPC - 1)) == 0)
            def _():
                load_lutc((g >> 4) << 8)

            in_copy(g, b).wait()

            @pl.when(p >= 1)
            def _():
                out_copy(g - 2, b).wait()

            ibuf = ibufs[b]
            obuf = obufs[b]

            for r in range(BR):
                @plsc.parallel_loop(0, W // 4, step=L, unroll=8)
                def _(k, ibuf=ibuf, obuf=obuf, r=r):
                    wv = ibuf[pl.ds(r * (W // 4) + k, L)]
                    b0 = wv & 255
                    b1 = lax.shift_right_logical(wv, 8) & 255
                    b2 = lax.shift_right_logical(wv, 16) & 255
                    b3 = lax.shift_right_logical(wv, 24)
                    i0 = k * 4
                    obuf[r, pl.ds(i0, L)] = plsc.load_gather(lutc, [b0])
                    obuf[r, pl.ds(i0 + L, L)] = plsc.load_gather(lutc, [b1])
                    obuf[r, pl.ds(i0 + 2 * L, L)] = plsc.load_gather(
                        lutc, [b2])
                    obuf[r, pl.ds(i0 + 3 * L, L)] = plsc.load_gather(
                        lutc, [b3])

            out_copy(g, b).start()

            @pl.when(p < (G // 2) - 1)
            def _():
                in_copy(g + 2, b).start()
        return 0

    lax.fori_loop(0, G // 2, chunk_body, 0)
    out_copy(G - 2, 0).wait()
    out_copy(G - 1, 1).wait()


def kernel(x):
    ph, bins = _hist_kernel(x)
    return _apply_kernel(bins, ph)
